# baseline (speedup 1.0000x reference)
import jax
import jax.numpy as jnp
from jax.experimental import pallas as pl
from jax.experimental.pallas import tpu as pltpu

_TILE = 512


def _fused(x_hbm, w_ref, pb_ref, out_hbm, xb_ref, ob_ref, insem, outsem):
    m = pb_ref.shape[1]
    t = _TILE
    w = w_ref[...]
    pb = pb_ref[...]

    nt = x_hbm.shape[0] // t
    cins = [pltpu.make_async_copy(x_hbm.at[pl.ds(j * t, t)], xb_ref.at[j],
                                  insem.at[j]) for j in range(nt)]
    for c in cins:
        c.start()

    mem2 = w[2 * m:3 * m]
    dcat = (w[0:2 * m] - jnp.concatenate([mem2, mem2], axis=0)
            ).astype(jnp.bfloat16)
    base = jnp.sum(mem2, axis=0)[None, :]

    def tile(buf, prev):
        y = jax.lax.dot_general(buf, w, (((1,), (1,)), ((), ())),
                                preferred_element_type=jnp.float32)
        full01 = jnp.concatenate([prev, y[:, 0:2 * m]], axis=0)
        s0 = full01[0:t, 0:m] + pb[0][None, :]
        s1 = full01[1:t + 1, m:2 * m] + pb[1][None, :]
        s2 = y[:, 2 * m:3 * m] + pb[2][None, :]
        o0 = (s0 >= s1) & (s0 >= s2)
        o1 = jnp.logical_not(o0) & (s1 >= s2)
        f = jnp.concatenate([o0.astype(jnp.bfloat16),
                             o1.astype(jnp.bfloat16)], axis=1)
        out = jax.lax.dot_general(f, dcat, (((1,), (0,)), ((), ())),
                                  preferred_element_type=jnp.float32)
        return out + base, y[t - 2:t, 0:2 * m]

    carry = jnp.zeros((2, 2 * m), jnp.float32)
    couts = []
    for j in range(nt):
        cins[j].wait()
        outj, carry = tile(xb_ref[j], carry)
        ob_ref[j] = outj
        c = pltpu.make_async_copy(ob_ref.at[j], out_hbm.at[pl.ds(j * t, t)],
                                  outsem.at[j])
        c.start()
        couts.append(c)
    for c in couts:
        c.wait()


def kernel(x, memory, pos_bias):
    b, l, d = x.shape
    m, n = pos_bias.shape
    w = memory.transpose(1, 0, 2).reshape(n * m, d)
    pb_t = pos_bias.T
    out = pl.pallas_call(
        _fused,
        in_specs=[
            pl.BlockSpec(memory_space=pltpu.MemorySpace.HBM),
            pl.BlockSpec(memory_space=pltpu.MemorySpace.VMEM),
            pl.BlockSpec(memory_space=pltpu.MemorySpace.VMEM),
        ],
        out_specs=pl.BlockSpec(memory_space=pltpu.MemorySpace.HBM),
        scratch_shapes=[pltpu.VMEM((l // _TILE, _TILE, d), jnp.float32),
                        pltpu.VMEM((l // _TILE, _TILE, d), jnp.float32),
                        pltpu.SemaphoreType.DMA((l // _TILE,)),
                        pltpu.SemaphoreType.DMA((l // _TILE,))],
        out_shape=jax.ShapeDtypeStruct((l, d), jnp.float32),
    )(x[0], w, pb_t)
    return out[None]


# speedup vs baseline: 1.1376x; 1.1376x over previous
import jax
import jax.numpy as jnp
from jax.experimental import pallas as pl
from jax.experimental.pallas import tpu as pltpu

_TILE = 1024


def _fused(x_ref, xb_ref, w_ref, pb_ref, out_ref):
    i = pl.program_id(0)
    t = x_ref.shape[0]
    m = pb_ref.shape[1]

    w = w_ref[...]
    y = jax.lax.dot_general(x_ref[...], w, (((1,), (1,)), ((), ())),
                            preferred_element_type=jnp.float32)
    pb = pb_ref[...]

    yb = jax.lax.dot_general(xb_ref[...], w, (((1,), (1,)), ((), ())),
                             preferred_element_type=jnp.float32)
    prev = jnp.where(i == 0, 0.0, yb[6:8, 0:2 * m])
    full01 = jnp.concatenate([prev, y[:, 0:2 * m]], axis=0)

    s0 = full01[0:t, 0:m] + pb[0][None, :]
    s1 = full01[1:t + 1, m:2 * m] + pb[1][None, :]
    s2 = y[:, 2 * m:3 * m] + pb[2][None, :]

    o0 = (s0 >= s1) & (s0 >= s2)
    o1 = jnp.logical_not(o0) & (s1 >= s2)
    f = jnp.concatenate([o0.astype(jnp.bfloat16),
                         o1.astype(jnp.bfloat16)], axis=1)

    mem2 = w[2 * m:3 * m]
    dcat = (w[0:2 * m] - jnp.concatenate([mem2, mem2], axis=0)
            ).astype(jnp.bfloat16)
    base = jnp.sum(mem2, axis=0)[None, :]

    out = jax.lax.dot_general(f, dcat, (((1,), (0,)), ((), ())),
                              preferred_element_type=jnp.float32)
    out_ref[...] = out + base


def kernel(x, memory, pos_bias):
    b, l, d = x.shape
    m, n = pos_bias.shape
    w = memory.transpose(1, 0, 2).reshape(n * m, d)
    pb_t = pos_bias.T
    nb = _TILE // 8
    out = pl.pallas_call(
        _fused,
        grid=(l // _TILE,),
        in_specs=[
            pl.BlockSpec((_TILE, d), lambda i: (i, 0)),
            pl.BlockSpec((8, d), lambda i: (jnp.maximum(i * nb - 1, 0), 0)),
            pl.BlockSpec((n * m, d), lambda i: (0, 0)),
            pl.BlockSpec((n, m), lambda i: (0, 0)),
        ],
        out_specs=pl.BlockSpec((_TILE, d), lambda i: (i, 0)),
        compiler_params=pltpu.CompilerParams(
            dimension_semantics=("parallel",)),
        out_shape=jax.ShapeDtypeStruct((l, d), jnp.float32),
    )(x[0], x[0], w, pb_t)
    return out[None]
